# Initial kernel scaffold; baseline (speedup 1.0000x reference)
import jax
import jax.numpy as jnp
from jax.experimental import pallas as pl


def kernel(inputs, random_rotations):
    raise NotImplementedError("write your pallas kernel here")


# speedup vs baseline: 3.3771x; 3.3771x over previous
import functools

import jax
import jax.numpy as jnp
from jax import lax
from jax.experimental import pallas as pl
from jax.experimental.pallas import tpu as pltpu
from jax.experimental.pallas import tpu_sc as plsc

_NBINS = 16
_K = 8
_CHUNK = 128


def _order_kernel(pts_ref, rot_ref, ord_ref, *, prec=lax.Precision.DEFAULT):
    n = pts_ref.shape[1]
    pts = pts_ref[0]
    rot = rot_ref[...]
    cm = lax.dot_general(pts, rot, (((1,), (0,)), ((), ())),
                         precision=prec)
    lane16 = lax.broadcasted_iota(jnp.int32, (n, _NBINS), 1)
    mx = jnp.max(cm, axis=1, keepdims=True)
    binc = jnp.min(jnp.where(cm == mx, lane16, _NBINS), axis=1,
                   keepdims=True)
    onehot = (lane16 == binc).astype(jnp.float32)

    nblk = n // _CHUNK
    r_lo = lax.broadcasted_iota(jnp.int32, (_CHUNK, _CHUNK), 0)
    c_lo = lax.broadcasted_iota(jnp.int32, (_CHUNK, _CHUNK), 1)
    lt = (c_lo < r_lo).astype(jnp.float32)
    prefix = jnp.zeros((1, _NBINS), jnp.float32)
    ranks = []
    for bb in range(nblk):
        oh_b = onehot[bb * _CHUNK:(bb + 1) * _CHUNK]
        e_b = lax.dot_general(lt, oh_b, (((1,), (0,)), ((), ())),
                              precision=lax.Precision.HIGHEST) + prefix
        ranks.append(jnp.sum(e_b * oh_b, axis=1, keepdims=True))
        prefix = prefix + jnp.sum(oh_b, axis=0, keepdims=True)
    counts = prefix
    r_i = jnp.concatenate(ranks, axis=0)
    ltmask = (lane16 < binc).astype(jnp.float32)
    p1 = jnp.sum(ltmask * counts, axis=1, keepdims=True)
    p = (p1 + r_i).astype(jnp.int32)

    base = pl.program_id(0) * n
    sb = 256
    for blk in range(n // sb):
        s_lane = lax.broadcasted_iota(jnp.int32, (n, sb), 1) + blk * sb
        eq = p == s_lane
        rowv = lax.broadcasted_iota(jnp.int32, (n, sb), 0).astype(jnp.float32)
        ob = jnp.sum(jnp.where(eq, rowv, 0.0), axis=0, keepdims=True)
        ord_ref[0, :, blk * sb:(blk + 1) * sb] = ob.astype(jnp.int32) + base


def _attn_kernel(sp_ref, inds_ref, vals_ref, *, prec=lax.Precision.DEFAULT,
                 cpb=4):
    dms = []
    for c in range(cpb):
        x = sp_ref[c * _CHUNK:(c + 1) * _CHUNK, :]
        sq = jnp.sum(x * x, axis=1, keepdims=True)
        nx = x * lax.rsqrt(jnp.maximum(sq, 1e-12))
        dms.append(lax.dot_general(nx, nx, (((1,), (1,)), ((), ())),
                                   precision=prec))
    dm = jnp.concatenate(dms, axis=0)
    nr = cpb * _CHUNK
    m = jnp.max(dm, axis=1, keepdims=True)
    e = jnp.exp(dm - m)
    sm = e / jnp.sum(e, axis=1, keepdims=True)
    lane = lax.broadcasted_iota(jnp.int32, (nr, _CHUNK), 1)
    work = sm
    icols, vcols = [], []
    for _ in range(_K):
        cur = jnp.max(work, axis=1, keepdims=True)
        arg = jnp.min(jnp.where(work == cur, lane, _CHUNK), axis=1,
                      keepdims=True)
        sel = lane == arg
        icols.append(arg)
        vcols.append(cur)
        work = jnp.where(sel, -1.0, work)
    inds_ref[...] = jnp.concatenate(icols, axis=1).reshape(cpb, _CHUNK, _K)
    vals_ref[...] = jnp.concatenate(vcols, axis=1).reshape(cpb, _CHUNK, _K)


def _gather_call(order_flat, pts_flat):
    nrows, d = pts_flat.shape
    mesh = plsc.VectorSubcoreMesh(core_axis_name="c", subcore_axis_name="s")
    rounds = nrows // (32 * _CHUNK)

    @functools.partial(
        pl.kernel, mesh=mesh,
        compiler_params=pltpu.CompilerParams(needs_layout_passes=False),
        out_type=jax.ShapeDtypeStruct((nrows, d), jnp.float32),
        scratch_types=[pltpu.VMEM((_CHUNK,), jnp.int32),
                       pltpu.VMEM((_CHUNK, d), jnp.float32),
                       pltpu.SemaphoreType.DMA],
    )
    def gather_k(ord_hbm, pts_hbm, out_hbm, idx_v, rows_v, sem):
        wid = lax.axis_index("s") * 2 + lax.axis_index("c")
        for r in range(rounds):
            blk = wid * rounds + r
            pltpu.sync_copy(ord_hbm.at[pl.ds(blk * _CHUNK, _CHUNK)], idx_v)
            pltpu.async_copy(pts_hbm.at[idx_v], rows_v, sem).wait()
            pltpu.sync_copy(rows_v, out_hbm.at[pl.ds(blk * _CHUNK, _CHUNK)])

    return gather_k(order_flat, pts_flat)


def _scatter_call(ord2d, inds2, vals2, n):
    nrows = ord2d.shape[0] * 16
    per_w = nrows // 32
    grp = 16
    ngrp = per_w // grp
    npairs = per_w // 2
    bw = n // per_w
    mesh = plsc.VectorSubcoreMesh(core_axis_name="c", subcore_axis_name="s")

    @functools.partial(
        pl.kernel, mesh=mesh,
        compiler_params=pltpu.CompilerParams(needs_layout_passes=False),
        out_type=jax.ShapeDtypeStruct((nrows, n), jnp.float32),
        scratch_types=[pltpu.VMEM((ngrp, grp), jnp.int32),
                       pltpu.VMEM((per_w,), jnp.int32),
                       pltpu.VMEM((npairs, 16), jnp.int32),
                       pltpu.VMEM((npairs, 16), jnp.float32),
                       pltpu.VMEM((grp, n), jnp.float32),
                       pltpu.SemaphoreType.DMA],
    )
    def scatter_k(ord_hbm, inds_hbm, vals_hbm, out_hbm,
                  idx2d, oloc, iv, vv, buf, sem):
        wid = lax.axis_index("s") * 2 + lax.axis_index("c")
        colbase = (wid // bw) * n
        pltpu.sync_copy(ord_hbm.at[pl.ds(wid * ngrp, ngrp)], idx2d)
        pltpu.sync_copy(inds_hbm.at[pl.ds(wid * npairs, npairs)], iv)
        pltpu.sync_copy(vals_hbm.at[pl.ds(wid * npairs, npairs)], vv)

        def mkloc(i, _):
            oloc[pl.ds(i * 16, 16)] = idx2d[i, :] - colbase
            return 0
        lax.fori_loop(0, per_w // 16, mkloc, 0)

        def zrow(t, _):
            buf[t // (n // 16), pl.ds((t % (n // 16)) * 16, 16)] = (
                jnp.zeros((16,), jnp.float32))
            return 0
        lax.fori_loop(0, grp * (n // 16), zrow, 0)

        lanei = lax.iota(jnp.int32, 16)
        rhalf = (lanei >= 8).astype(jnp.int32)
        zero16 = jnp.zeros((16,), jnp.float32)

        def do_group(g, _):
            for j in range(grp // 2):
                q = g * (grp // 2) + j
                gidx = iv[q, :] + (q // (_CHUNK // 2)) * _CHUNK
                colg = plsc.load_gather(oloc, [gidx])
                plsc.store_scatter(buf, [rhalf + 2 * j, colg], vv[q, :])
            pltpu.async_copy(buf, out_hbm.at[idx2d.at[g]], sem).wait()
            for j in range(grp // 2):
                q = g * (grp // 2) + j
                gidx = iv[q, :] + (q // (_CHUNK // 2)) * _CHUNK
                colg = plsc.load_gather(oloc, [gidx])
                plsc.store_scatter(buf, [rhalf + 2 * j, colg], zero16)
            return 0
        lax.fori_loop(0, ngrp, do_group, 0)

    return scatter_k(ord2d, inds2, vals2)


def _run(inputs, rot_ext):
    b, n, d = inputs.shape
    order3 = pl.pallas_call(
        _order_kernel,
        grid=(b,),
        in_specs=[pl.BlockSpec((1, n, d), lambda i: (i, 0, 0)),
                  pl.BlockSpec((d, _NBINS), lambda i: (0, 0))],
        out_specs=pl.BlockSpec((1, 1, n), lambda i: (i, 0, 0)),
        out_shape=jax.ShapeDtypeStruct((b, 1, n), jnp.int32),
    )(inputs, rot_ext)

    order_flat = order3.reshape(b * n)
    pts_flat = inputs.reshape(b * n, d)
    sorted_pts = _gather_call(order_flat, pts_flat)

    nch = (b * n) // _CHUNK
    cpb = 16
    inds, vals = pl.pallas_call(
        functools.partial(_attn_kernel, cpb=cpb),
        grid=(nch // cpb,),
        in_specs=[pl.BlockSpec((cpb * _CHUNK, d), lambda c: (c, 0))],
        out_specs=[pl.BlockSpec((cpb, _CHUNK, _K), lambda c: (c, 0, 0)),
                   pl.BlockSpec((cpb, _CHUNK, _K), lambda c: (c, 0, 0))],
        out_shape=[jax.ShapeDtypeStruct((nch, _CHUNK, _K), jnp.int32),
                   jax.ShapeDtypeStruct((nch, _CHUNK, _K), jnp.float32)],
    )(sorted_pts)

    inds2 = inds.reshape(b * n // 2, 2 * _K)
    vals2 = vals.reshape(b * n // 2, 2 * _K)
    ord2d = order3.reshape(b * n // 16, 16)
    out_flat = _scatter_call(ord2d, inds2, vals2, n)
    return out_flat.reshape(b, n, n)


def kernel(inputs, random_rotations):
    b = inputs.shape[0]
    rot_ext = jnp.concatenate([random_rotations, -random_rotations], axis=1)
    devs = jax.devices()
    if len(devs) >= 2 and b % 2 == 0:
        import numpy as np
        from jax.sharding import Mesh, PartitionSpec as P
        mesh = Mesh(np.asarray(devs[:2]), ("dv",))
        return jax.shard_map(_run, mesh=mesh,
                             in_specs=(P("dv"), P()),
                             out_specs=P("dv"),
                             check_vma=False)(inputs, rot_ext)
    return _run(inputs, rot_ext)


# speedup vs baseline: 11.4342x; 3.3858x over previous
import functools

import jax
import jax.numpy as jnp
from jax import lax
from jax.experimental import pallas as pl
from jax.experimental.pallas import tpu as pltpu
from jax.experimental.pallas import tpu_sc as plsc

_NBINS = 16
_K = 8
_CHUNK = 128


def _order_kernel(pts_ref, rot_ref, ord_ref, *, bbase,
                  prec=lax.Precision.DEFAULT):
    n = pts_ref.shape[1]
    pts = pts_ref[0]
    rot = rot_ref[...]
    cm = lax.dot_general(pts, rot, (((1,), (0,)), ((), ())),
                         precision=prec)
    lane16 = lax.broadcasted_iota(jnp.int32, (n, _NBINS), 1)
    mx = jnp.max(cm, axis=1, keepdims=True)
    binc = jnp.min(jnp.where(cm == mx, lane16, _NBINS), axis=1,
                   keepdims=True)
    onehot = (lane16 == binc).astype(jnp.float32)

    nblk = n // _CHUNK
    r_lo = lax.broadcasted_iota(jnp.int32, (_CHUNK, _CHUNK), 0)
    c_lo = lax.broadcasted_iota(jnp.int32, (_CHUNK, _CHUNK), 1)
    lt = (c_lo < r_lo).astype(jnp.float32)
    prefix = jnp.zeros((1, _NBINS), jnp.float32)
    ranks = []
    for bb in range(nblk):
        oh_b = onehot[bb * _CHUNK:(bb + 1) * _CHUNK]
        e_b = lax.dot_general(lt, oh_b, (((1,), (0,)), ((), ())),
                              precision=lax.Precision.HIGHEST) + prefix
        ranks.append(jnp.sum(e_b * oh_b, axis=1, keepdims=True))
        prefix = prefix + jnp.sum(oh_b, axis=0, keepdims=True)
    counts = prefix
    r_i = jnp.concatenate(ranks, axis=0)
    ltmask = (lane16 < binc).astype(jnp.float32)
    p1 = jnp.sum(ltmask * counts, axis=1, keepdims=True)
    p = (p1 + r_i).astype(jnp.int32)

    base = (pl.program_id(0) + bbase) * n
    sb = 256
    for blk in range(n // sb):
        s_lane = lax.broadcasted_iota(jnp.int32, (n, sb), 1) + blk * sb
        eq = p == s_lane
        rowv = lax.broadcasted_iota(jnp.int32, (n, sb), 0).astype(jnp.float32)
        ob = jnp.sum(jnp.where(eq, rowv, 0.0), axis=0, keepdims=True)
        ord_ref[0, :, blk * sb:(blk + 1) * sb] = ob.astype(jnp.int32) + base


def _attn_kernel(sp_ref, inds_ref, vals_ref, *, prec=lax.Precision.DEFAULT,
                 cpb=16):
    dms = []
    for c in range(cpb):
        x = sp_ref[c * _CHUNK:(c + 1) * _CHUNK, :]
        sq = jnp.sum(x * x, axis=1, keepdims=True)
        nx = x * lax.rsqrt(jnp.maximum(sq, 1e-12))
        dms.append(lax.dot_general(nx, nx, (((1,), (1,)), ((), ())),
                                   precision=prec))
    dm = jnp.concatenate(dms, axis=0)
    nr = cpb * _CHUNK
    m = jnp.max(dm, axis=1, keepdims=True)
    e = jnp.exp(dm - m)
    sm = e / jnp.sum(e, axis=1, keepdims=True)
    lane = lax.broadcasted_iota(jnp.int32, (nr, _CHUNK), 1)
    work = sm
    icols, vcols = [], []
    for _ in range(_K):
        cur = jnp.max(work, axis=1, keepdims=True)
        arg = jnp.min(jnp.where(work == cur, lane, _CHUNK), axis=1,
                      keepdims=True)
        sel = lane == arg
        icols.append(arg)
        vcols.append(cur)
        work = jnp.where(sel, -1.0, work)
    inds_ref[...] = jnp.concatenate(icols, axis=1).reshape(cpb, _CHUNK, _K)
    vals_ref[...] = jnp.concatenate(vcols, axis=1).reshape(cpb, _CHUNK, _K)


def _gather_call(order_flat, pts_flat):
    nrows = order_flat.shape[0]
    d = pts_flat.shape[1]
    mesh = plsc.VectorSubcoreMesh(core_axis_name="c", subcore_axis_name="s")
    per_w = nrows // 32

    @functools.partial(
        pl.kernel, mesh=mesh,
        compiler_params=pltpu.CompilerParams(needs_layout_passes=False),
        out_type=jax.ShapeDtypeStruct((nrows, d), jnp.float32),
        scratch_types=[pltpu.VMEM((per_w,), jnp.int32),
                       pltpu.VMEM((per_w, d), jnp.float32),
                       pltpu.SemaphoreType.DMA],
    )
    def gather_k(ord_hbm, pts_hbm, out_hbm, idx_v, rows_v, sem):
        wid = lax.axis_index("s") * 2 + lax.axis_index("c")
        base = wid * per_w
        pltpu.sync_copy(ord_hbm.at[pl.ds(base, per_w)], idx_v)
        pltpu.async_copy(pts_hbm.at[idx_v], rows_v, sem).wait()
        pltpu.sync_copy(rows_v, out_hbm.at[pl.ds(base, per_w)])

    return gather_k(order_flat, pts_flat)


def _scatter_into(out_ref, ord2d, inds2, vals2, n, hbase):
    nrows = ord2d.shape[0] * 16
    per_w = nrows // 32
    grp = 16
    ngrp = per_w // grp
    npairs = per_w // 2
    bw = n // per_w
    mesh = plsc.VectorSubcoreMesh(core_axis_name="c", subcore_axis_name="s")

    @functools.partial(
        pl.kernel, mesh=mesh,
        compiler_params=pltpu.CompilerParams(needs_layout_passes=False),
        out_type=(),
        scratch_types=[pltpu.VMEM((ngrp, grp), jnp.int32),
                       pltpu.VMEM((per_w,), jnp.int32),
                       pltpu.VMEM((npairs, 16), jnp.int32),
                       pltpu.VMEM((npairs, 16), jnp.float32),
                       pltpu.VMEM((grp, n), jnp.float32),
                       pltpu.SemaphoreType.DMA],
    )
    def scatter_k(ord_hbm, inds_hbm, vals_hbm, out_hbm,
                  idx2d, oloc, iv, vv, buf, sem):
        wid = lax.axis_index("s") * 2 + lax.axis_index("c")
        colbase = (wid // bw + hbase) * n
        pltpu.sync_copy(ord_hbm.at[pl.ds(wid * ngrp, ngrp)], idx2d)
        pltpu.sync_copy(inds_hbm.at[pl.ds(wid * npairs, npairs)], iv)
        pltpu.sync_copy(vals_hbm.at[pl.ds(wid * npairs, npairs)], vv)

        def mkloc(i, _):
            oloc[pl.ds(i * 16, 16)] = idx2d[i, :] - colbase
            return 0
        lax.fori_loop(0, per_w // 16, mkloc, 0)

        def zrow(t, _):
            buf[t // (n // 16), pl.ds((t % (n // 16)) * 16, 16)] = (
                jnp.zeros((16,), jnp.float32))
            return 0
        lax.fori_loop(0, grp * (n // 16), zrow, 0)

        lanei = lax.iota(jnp.int32, 16)
        rhalf = (lanei >= 8).astype(jnp.int32)
        zero16 = jnp.zeros((16,), jnp.float32)

        def do_group(g, _):
            for j in range(grp // 2):
                q = g * (grp // 2) + j
                gidx = iv[q, :] + (q // (_CHUNK // 2)) * _CHUNK
                colg = plsc.load_gather(oloc, [gidx])
                plsc.store_scatter(buf, [rhalf + 2 * j, colg], vv[q, :])
            pltpu.async_copy(buf, out_hbm.at[idx2d.at[g]], sem).wait()
            for j in range(grp // 2):
                q = g * (grp // 2) + j
                gidx = iv[q, :] + (q // (_CHUNK // 2)) * _CHUNK
                colg = plsc.load_gather(oloc, [gidx])
                plsc.store_scatter(buf, [rhalf + 2 * j, colg], zero16)
            return 0
        lax.fori_loop(0, ngrp, do_group, 0)

    scatter_k(ord2d, inds2, vals2, out_ref)


def kernel(inputs, random_rotations):
    b, n, d = inputs.shape
    rot_ext = jnp.concatenate([random_rotations, -random_rotations], axis=1)
    pts_flat = inputs.reshape(b * n, d)
    hb = b // 2
    out_ref = jax.new_ref(jnp.zeros((b * n, n), jnp.float32))
    for h in range(2):
        order3 = pl.pallas_call(
            functools.partial(_order_kernel, bbase=h * hb),
            grid=(hb,),
            in_specs=[pl.BlockSpec((1, n, d), lambda i, h=h: (i + h * hb, 0, 0)),
                      pl.BlockSpec((d, _NBINS), lambda i: (0, 0))],
            out_specs=pl.BlockSpec((1, 1, n), lambda i: (i, 0, 0)),
            out_shape=jax.ShapeDtypeStruct((hb, 1, n), jnp.int32),
        )(inputs, rot_ext)

        order_flat = order3.reshape(hb * n)
        sorted_pts = _gather_call(order_flat, pts_flat)

        nch = (hb * n) // _CHUNK
        cpb = 16
        inds, vals = pl.pallas_call(
            functools.partial(_attn_kernel, cpb=cpb),
            grid=(nch // cpb,),
            in_specs=[pl.BlockSpec((cpb * _CHUNK, d), lambda c: (c, 0))],
            out_specs=[pl.BlockSpec((cpb, _CHUNK, _K), lambda c: (c, 0, 0)),
                       pl.BlockSpec((cpb, _CHUNK, _K), lambda c: (c, 0, 0))],
            out_shape=[jax.ShapeDtypeStruct((nch, _CHUNK, _K), jnp.int32),
                       jax.ShapeDtypeStruct((nch, _CHUNK, _K), jnp.float32)],
        )(sorted_pts)

        inds2 = inds.reshape(hb * n // 2, 2 * _K)
        vals2 = vals.reshape(hb * n // 2, 2 * _K)
        ord2d = order3.reshape(hb * n // 16, 16)
        _scatter_into(out_ref, ord2d, inds2, vals2, n, h * hb)
    return out_ref[...].reshape(b, n, n)


# speedup vs baseline: 12.7816x; 1.1178x over previous
import functools

import jax
import jax.numpy as jnp
from jax import lax
from jax.experimental import pallas as pl
from jax.experimental.pallas import tpu as pltpu
from jax.experimental.pallas import tpu_sc as plsc

_NBINS = 16
_K = 8
_CHUNK = 128


def _order_kernel(pts_ref, rot_ref, ord_ref, *, bbase,
                  prec=lax.Precision.DEFAULT):
    n = pts_ref.shape[1]
    pts = pts_ref[0]
    rot = rot_ref[...]
    cm = lax.dot_general(pts, rot, (((1,), (0,)), ((), ())),
                         precision=prec)
    lane16 = lax.broadcasted_iota(jnp.int32, (n, _NBINS), 1)
    mx = jnp.max(cm, axis=1, keepdims=True)
    binc = jnp.min(jnp.where(cm == mx, lane16, _NBINS), axis=1,
                   keepdims=True)
    onehot = (lane16 == binc).astype(jnp.float32)

    nblk = n // _CHUNK
    r_lo = lax.broadcasted_iota(jnp.int32, (_CHUNK, _CHUNK), 0)
    c_lo = lax.broadcasted_iota(jnp.int32, (_CHUNK, _CHUNK), 1)
    lt = (c_lo < r_lo).astype(jnp.float32)
    prefix = jnp.zeros((1, _NBINS), jnp.float32)
    ranks = []
    for bb in range(nblk):
        oh_b = onehot[bb * _CHUNK:(bb + 1) * _CHUNK]
        e_b = lax.dot_general(lt, oh_b, (((1,), (0,)), ((), ())),
                              precision=lax.Precision.HIGHEST) + prefix
        ranks.append(jnp.sum(e_b * oh_b, axis=1, keepdims=True))
        prefix = prefix + jnp.sum(oh_b, axis=0, keepdims=True)
    counts = prefix
    r_i = jnp.concatenate(ranks, axis=0)
    ltmask = (lane16 < binc).astype(jnp.float32)
    p1 = jnp.sum(ltmask * counts, axis=1, keepdims=True)
    p = (p1 + r_i).astype(jnp.int32)

    base = (pl.program_id(0) + bbase) * n
    sb = 256
    for blk in range(n // sb):
        s_lane = lax.broadcasted_iota(jnp.int32, (n, sb), 1) + blk * sb
        eq = p == s_lane
        rowv = lax.broadcasted_iota(jnp.int32, (n, sb), 0).astype(jnp.float32)
        ob = jnp.sum(jnp.where(eq, rowv, 0.0), axis=0, keepdims=True)
        ord_ref[0, :, blk * sb:(blk + 1) * sb] = ob.astype(jnp.int32) + base


def _attn_kernel(sp_ref, inds_ref, vals_ref, *, prec=lax.Precision.DEFAULT,
                 cpb=16):
    dms = []
    for c in range(cpb):
        x = sp_ref[c * _CHUNK:(c + 1) * _CHUNK, :]
        sq = jnp.sum(x * x, axis=1, keepdims=True)
        nx = x * lax.rsqrt(jnp.maximum(sq, 1e-12))
        dms.append(lax.dot_general(nx, nx, (((1,), (1,)), ((), ())),
                                   precision=prec))
    dm = jnp.concatenate(dms, axis=0)
    nr = cpb * _CHUNK
    m = jnp.max(dm, axis=1, keepdims=True)
    e = jnp.exp(dm - m)
    sm = e / jnp.sum(e, axis=1, keepdims=True)
    lane = lax.broadcasted_iota(jnp.int32, (nr, _CHUNK), 1)
    work = sm
    icols, vcols = [], []
    for _ in range(_K):
        cur = jnp.max(work, axis=1, keepdims=True)
        arg = jnp.min(jnp.where(work == cur, lane, _CHUNK), axis=1,
                      keepdims=True)
        sel = lane == arg
        icols.append(arg)
        vcols.append(cur)
        work = jnp.where(sel, -1.0, work)
    inds_ref[...] = jnp.concatenate(icols, axis=1).reshape(cpb, _CHUNK, _K)
    vals_ref[...] = jnp.concatenate(vcols, axis=1).reshape(cpb, _CHUNK, _K)


def _gather_call(order_flat, pts_flat):
    nrows = order_flat.shape[0]
    d = pts_flat.shape[1]
    mesh = plsc.VectorSubcoreMesh(core_axis_name="c", subcore_axis_name="s")
    per_w = nrows // 32

    @functools.partial(
        pl.kernel, mesh=mesh,
        compiler_params=pltpu.CompilerParams(needs_layout_passes=False),
        out_type=jax.ShapeDtypeStruct((nrows, d), jnp.float32),
        scratch_types=[pltpu.VMEM((per_w,), jnp.int32),
                       pltpu.VMEM((per_w, d), jnp.float32),
                       pltpu.SemaphoreType.DMA],
    )
    def gather_k(ord_hbm, pts_hbm, out_hbm, idx_v, rows_v, sem):
        wid = lax.axis_index("s") * 2 + lax.axis_index("c")
        base = wid * per_w
        pltpu.sync_copy(ord_hbm.at[pl.ds(base, per_w)], idx_v)
        pltpu.async_copy(pts_hbm.at[idx_v], rows_v, sem).wait()
        pltpu.sync_copy(rows_v, out_hbm.at[pl.ds(base, per_w)])

    return gather_k(order_flat, pts_flat)


def _scatter_call(ord2d, inds2, vals2, n):
    nrows = ord2d.shape[0] * 16
    per_w = nrows // 32
    grp = 16
    ngrp = per_w // grp
    npairs = per_w // 2
    bw = n // per_w
    mesh = plsc.VectorSubcoreMesh(core_axis_name="c", subcore_axis_name="s")

    @functools.partial(
        pl.kernel, mesh=mesh,
        compiler_params=pltpu.CompilerParams(needs_layout_passes=False),
        out_type=jax.ShapeDtypeStruct((nrows, n), jnp.float32),
        scratch_types=[pltpu.VMEM((ngrp, grp), jnp.int32),
                       pltpu.VMEM((per_w,), jnp.int32),
                       pltpu.VMEM((npairs, 16), jnp.int32),
                       pltpu.VMEM((npairs, 16), jnp.float32),
                       pltpu.VMEM((grp, n), jnp.float32),
                       pltpu.SemaphoreType.DMA],
    )
    def scatter_k(ord_hbm, inds_hbm, vals_hbm, out_hbm,
                  idx2d, oloc, iv, vv, buf, sem):
        wid = lax.axis_index("s") * 2 + lax.axis_index("c")
        colbase = (wid // bw) * n
        pltpu.sync_copy(ord_hbm.at[pl.ds(wid * ngrp, ngrp)], idx2d)
        pltpu.sync_copy(inds_hbm.at[pl.ds(wid * npairs, npairs)], iv)
        pltpu.sync_copy(vals_hbm.at[pl.ds(wid * npairs, npairs)], vv)

        def mkloc(i, _):
            oloc[pl.ds(i * 16, 16)] = idx2d[i, :] - colbase
            return 0
        lax.fori_loop(0, per_w // 16, mkloc, 0)

        def zrow(t, _):
            buf[t // (n // 16), pl.ds((t % (n // 16)) * 16, 16)] = (
                jnp.zeros((16,), jnp.float32))
            return 0
        lax.fori_loop(0, grp * (n // 16), zrow, 0)

        lanei = lax.iota(jnp.int32, 16)
        rhalf = (lanei >= 8).astype(jnp.int32)
        zero16 = jnp.zeros((16,), jnp.float32)

        def do_group(g, _):
            for j in range(grp // 2):
                q = g * (grp // 2) + j
                gidx = iv[q, :] + (q // (_CHUNK // 2)) * _CHUNK
                colg = plsc.load_gather(oloc, [gidx])
                plsc.store_scatter(buf, [rhalf + 2 * j, colg], vv[q, :])
            pltpu.async_copy(buf, out_hbm.at[idx2d.at[g]], sem).wait()
            for j in range(grp // 2):
                q = g * (grp // 2) + j
                gidx = iv[q, :] + (q // (_CHUNK // 2)) * _CHUNK
                colg = plsc.load_gather(oloc, [gidx])
                plsc.store_scatter(buf, [rhalf + 2 * j, colg], zero16)
            return 0
        lax.fori_loop(0, ngrp, do_group, 0)

    return scatter_k(ord2d, inds2, vals2)


def kernel(inputs, random_rotations):
    b, n, d = inputs.shape
    rot_ext = jnp.concatenate([random_rotations, -random_rotations], axis=1)
    pts_flat = inputs.reshape(b * n, d)
    hb = b // 2
    parts = []
    for h in range(2):
        order3 = pl.pallas_call(
            functools.partial(_order_kernel, bbase=h * hb),
            grid=(hb,),
            in_specs=[pl.BlockSpec((1, n, d), lambda i, h=h: (i + h * hb, 0, 0)),
                      pl.BlockSpec((d, _NBINS), lambda i: (0, 0))],
            out_specs=pl.BlockSpec((1, 1, n), lambda i: (i, 0, 0)),
            out_shape=jax.ShapeDtypeStruct((hb, 1, n), jnp.int32),
        )(inputs, rot_ext)

        order_flat = order3.reshape(hb * n)
        sorted_pts = _gather_call(order_flat, pts_flat)

        nch = (hb * n) // _CHUNK
        cpb = 16
        inds, vals = pl.pallas_call(
            functools.partial(_attn_kernel, cpb=cpb),
            grid=(nch // cpb,),
            in_specs=[pl.BlockSpec((cpb * _CHUNK, d), lambda c: (c, 0))],
            out_specs=[pl.BlockSpec((cpb, _CHUNK, _K), lambda c: (c, 0, 0)),
                       pl.BlockSpec((cpb, _CHUNK, _K), lambda c: (c, 0, 0))],
            out_shape=[jax.ShapeDtypeStruct((nch, _CHUNK, _K), jnp.int32),
                       jax.ShapeDtypeStruct((nch, _CHUNK, _K), jnp.float32)],
        )(sorted_pts)

        inds2 = inds.reshape(hb * n // 2, 2 * _K)
        vals2 = vals.reshape(hb * n // 2, 2 * _K)
        ord2d = order3.reshape(hb * n // 16, 16)
        parts.append((ord2d, inds2, vals2))
    ord2d = jnp.concatenate([p[0] for p in parts], axis=0)
    inds2 = jnp.concatenate([p[1] for p in parts], axis=0)
    vals2 = jnp.concatenate([p[2] for p in parts], axis=0)
    out_flat = _scatter_call(ord2d, inds2, vals2, n)
    return out_flat.reshape(b, n, n)
